# baseline (speedup 1.0000x reference)
import functools

import jax
import jax.numpy as jnp
from jax import lax
from jax.experimental import pallas as pl
from jax.experimental.pallas import tpu as pltpu
from jax.experimental.pallas import tpu_sc as plsc

_N = 10000
_E = 320000
_D = 128
_NCLS = 47
_W2 = 48

_NC = 2
_NS = 16
_NW = _NC * _NS
_CH = 128
_K = 80
_G = 8
_NG = _K // _G
_E_PAD = _NW * _K * _CH
_N_PAD = 10016
_RPT = _N_PAD // _NS


def _sc_scatter(width: int, with_counts: bool, extra: int):
    mesh = plsc.VectorSubcoreMesh(core_axis_name="c", subcore_axis_name="s",
                                  num_cores=_NC, num_subcores=_NS)
    cparams = pltpu.CompilerParams(use_tc_tiling_on_sc=False)
    out_type = [jax.ShapeDtypeStruct((_NC, _N_PAD, width), jnp.float32)]
    if with_counts:
        out_type.append(jax.ShapeDtypeStruct((_NC, _N_PAD, 16), jnp.float32))
    scratch = [
        pltpu.VMEM((_G, _CH), jnp.int32),
        pltpu.VMEM((_G, _CH), jnp.int32),
    ]
    nbuf = 2 if with_counts else 4
    scratch += [pltpu.VMEM((_CH, width), jnp.float32)] * nbuf
    scratch += [
        pltpu.VMEM_SHARED((_N_PAD, width), jnp.float32),
        pltpu.SemaphoreType.DMA,
        pltpu.SemaphoreType.DMA,
    ]
    if with_counts:
        scratch += [
            pltpu.VMEM((_CH, 16), jnp.float32),
            pltpu.VMEM_SHARED((_N_PAD, 16), jnp.float32),
            pltpu.SemaphoreType.DMA,
        ]

    def _group_body(feats, src_v, dst_v, bufs, acc_sh, sem_g, sem_s,
                    ones_v, cnt_sh, sem_c):
        nb = len(bufs)
        dg = [None] * _G
        ds = [None] * _G
        dc = [None] * _G
        for j in range(min(nb - 1, _G)):
            dg[j] = pltpu.async_copy(feats.at[src_v.at[j]], bufs[j % nb],
                                     sem_g)
        for j in range(_G):
            if j >= 1:
                ds[j - 1].wait()
                if with_counts:
                    dc[j - 1].wait()
            nxt = j + nb - 1
            if nxt < _G:
                dg[nxt] = pltpu.async_copy(feats.at[src_v.at[nxt]],
                                           bufs[nxt % nb], sem_g)
            dg[j].wait()
            ds[j] = pltpu.async_copy(bufs[j % nb], acc_sh.at[dst_v.at[j]],
                                     sem_s, add=True)
            if with_counts:
                dc[j] = pltpu.async_copy(ones_v, cnt_sh.at[dst_v.at[j]],
                                         sem_c, add=True)
        ds[_G - 1].wait()
        if with_counts:
            dc[_G - 1].wait()

    if with_counts:
        @functools.partial(pl.kernel, out_type=out_type, mesh=mesh,
                           scratch_types=scratch, compiler_params=cparams)
        def kern(feats, src3, dst3, zrow, zcnt,
                 s_out, c_out, src_v, dst_v, *rest):
            bufs = rest[:nbuf]
            acc_sh, sem_g, sem_s, ones_v, cnt_sh, sem_c = rest[nbuf:]
            cid = lax.axis_index("c")
            sid = lax.axis_index("s")
            wid_own = sid * _NC + cid
            wid_par = sid * _NC + (1 - cid)
            rows = pl.ds(sid * _RPT, _RPT)
            pltpu.sync_copy(zrow, acc_sh.at[rows])
            pltpu.sync_copy(zcnt, cnt_sh.at[rows])
            for i in range(_CH):
                ones_v[i, :] = jnp.full((16,), 1.0, jnp.float32)
            plsc.subcore_barrier()

            def run_group(w, g):
                pltpu.sync_copy(src3.at[w, pl.ds(g * _G, _G)], src_v)
                pltpu.sync_copy(dst3.at[w, pl.ds(g * _G, _G)], dst_v)
                _group_body(feats, src_v, dst_v, bufs, acc_sh,
                            sem_g, sem_s, ones_v, cnt_sh, sem_c)

            @pl.when(cid == 0)
            def _():
                @pl.loop(0, _NG)
                def _(g):
                    run_group(wid_own, g)

                @pl.loop(_NG - extra, _NG)
                def _(g):
                    run_group(wid_par, g)

            @pl.when(cid == 1)
            def _():
                @pl.loop(0, _NG - extra)
                def _(g):
                    run_group(wid_own, g)

            plsc.subcore_barrier()
            pltpu.sync_copy(acc_sh.at[rows], s_out.at[cid, rows])
            pltpu.sync_copy(cnt_sh.at[rows], c_out.at[cid, rows])
        return kern

    @functools.partial(pl.kernel, out_type=out_type, mesh=mesh,
                       scratch_types=scratch, compiler_params=cparams)
    def kern2(feats, src3, dst3, zrow,
              s_out, src_v, dst_v, *rest):
        bufs = rest[:nbuf]
        acc_sh, sem_g, sem_s = rest[nbuf:]
        cid = lax.axis_index("c")
        sid = lax.axis_index("s")
        wid_own = sid * _NC + cid
        wid_par = sid * _NC + (1 - cid)
        rows = pl.ds(sid * _RPT, _RPT)
        pltpu.sync_copy(zrow, acc_sh.at[rows])
        plsc.subcore_barrier()

        def run_group(w, g):
            pltpu.sync_copy(src3.at[w, pl.ds(g * _G, _G)], src_v)
            pltpu.sync_copy(dst3.at[w, pl.ds(g * _G, _G)], dst_v)
            _group_body(feats, src_v, dst_v, bufs, acc_sh,
                        sem_g, sem_s, None, None, None)

        @pl.when(cid == 0)
        def _():
            @pl.loop(0, _NG)
            def _(g):
                run_group(wid_own, g)

            @pl.loop(_NG - extra, _NG)
            def _(g):
                run_group(wid_par, g)

        @pl.when(cid == 1)
        def _():
            @pl.loop(0, _NG - extra)
            def _(g):
                run_group(wid_own, g)

        plsc.subcore_barrier()
        pltpu.sync_copy(acc_sh.at[rows], s_out.at[cid, rows])
    return kern2


_sc_layer1 = _sc_scatter(_D, True, extra=6)
_sc_layer2 = _sc_scatter(_W2, False, extra=4)


def _tc_mid(s1, cnt, x, W1l, b1, W1r, W2lp, b2p, W2rp):
    blk = 1000
    grid = _N // blk

    def body(s_ref, c_ref, x_ref, w1l, b1r, w1r, w2l, b2r, w2r,
             p2_ref, r2_ref):
        ssum = s_ref[0] + s_ref[1]
        csum = jnp.maximum(c_ref[0] + c_ref[1], 1.0)[:, 0:1]
        mean1 = ssum / csum
        xb = x_ref[...]
        h1 = mean1 @ w1l[...].T + b1r[...] + xb @ w1r[...].T
        h1 = jnp.maximum(h1, 0.0)
        p2_ref[...] = h1 @ w2l[...].T
        r2_ref[...] = h1 @ w2r[...].T + b2r[...]

    return pl.pallas_call(
        body,
        grid=(grid,),
        in_specs=[
            pl.BlockSpec((_NC, blk, _D), lambda i: (0, i, 0)),
            pl.BlockSpec((_NC, blk, 16), lambda i: (0, i, 0)),
            pl.BlockSpec((blk, _D), lambda i: (i, 0)),
            pl.BlockSpec((_D, _D), lambda i: (0, 0)),
            pl.BlockSpec((1, _D), lambda i: (0, 0)),
            pl.BlockSpec((_D, _D), lambda i: (0, 0)),
            pl.BlockSpec((_W2, _D), lambda i: (0, 0)),
            pl.BlockSpec((1, _W2), lambda i: (0, 0)),
            pl.BlockSpec((_W2, _D), lambda i: (0, 0)),
        ],
        out_specs=[
            pl.BlockSpec((blk, _W2), lambda i: (i, 0)),
            pl.BlockSpec((blk, _W2), lambda i: (i, 0)),
        ],
        out_shape=[
            jax.ShapeDtypeStruct((_N, _W2), jnp.float32),
            jax.ShapeDtypeStruct((_N, _W2), jnp.float32),
        ],
    )(s1, cnt, x, W1l, b1, W1r, W2lp, b2p, W2rp)


def _tc_final(s2, cnt, r2):
    blk = 1000
    grid = _N // blk

    def body(s_ref, c_ref, r_ref, o_ref):
        ssum = s_ref[0] + s_ref[1]
        csum = jnp.maximum(c_ref[0] + c_ref[1], 1.0)[:, 0:1]
        o_ref[...] = ssum / csum + r_ref[...]

    return pl.pallas_call(
        body,
        grid=(grid,),
        in_specs=[
            pl.BlockSpec((_NC, blk, _W2), lambda i: (0, i, 0)),
            pl.BlockSpec((_NC, blk, 16), lambda i: (0, i, 0)),
            pl.BlockSpec((blk, _W2), lambda i: (i, 0)),
        ],
        out_specs=pl.BlockSpec((blk, _W2), lambda i: (i, 0)),
        out_shape=jax.ShapeDtypeStruct((_N, _W2), jnp.float32),
    )(s2, cnt, r2)


def kernel(x, edge_index, W1l, b1, W1r, W2l, b2, W2r):
    src = edge_index[0].astype(jnp.int32)
    dst = edge_index[1].astype(jnp.int32)
    pad = _E_PAD - _E
    src3 = jnp.concatenate([src, jnp.zeros((pad,), jnp.int32)]
                           ).reshape(_NW, _K, _CH)
    dst3 = jnp.concatenate([dst, jnp.full((pad,), _N, jnp.int32)]
                           ).reshape(_NW, _K, _CH)
    zrow = jnp.zeros((_RPT, _D), jnp.float32)
    zrow2 = jnp.zeros((_RPT, _W2), jnp.float32)
    zcnt = jnp.zeros((_RPT, 16), jnp.float32)

    s1, cnt = _sc_layer1(x, src3, dst3, zrow, zcnt)

    W2lp = jnp.zeros((_W2, _D), jnp.float32).at[:_NCLS].set(W2l)
    W2rp = jnp.zeros((_W2, _D), jnp.float32).at[:_NCLS].set(W2r)
    b2p = jnp.zeros((1, _W2), jnp.float32).at[0, :_NCLS].set(b2)

    p2, r2 = _tc_mid(s1, cnt, x, W1l, b1.reshape(1, _D), W1r,
                     W2lp, b2p, W2rp)

    [s2] = _sc_layer2(p2, src3, dst3, zrow2)

    out = _tc_final(s2, cnt, r2)
    return out[:, :_NCLS]


# speedup vs baseline: 1.1407x; 1.1407x over previous
import functools

import jax
import jax.numpy as jnp
from jax import lax
from jax.experimental import pallas as pl
from jax.experimental.pallas import tpu as pltpu
from jax.experimental.pallas import tpu_sc as plsc

_N = 10000
_E = 320000
_D = 128
_NCLS = 47
_W2 = 48

_NC = 2
_NS = 16
_NW = _NC * _NS
_CH = 128
_KT = 160
_G = 8
_NGT = _KT // _G
_NGF = 15
_FAST_CID = 1
_E_PAD = _NS * _KT * _CH
_N_PAD = 10016
_RPT = _N_PAD // _NS


def _sc_scatter(width: int, with_counts: bool):
    mesh = plsc.VectorSubcoreMesh(core_axis_name="c", subcore_axis_name="s",
                                  num_cores=_NC, num_subcores=_NS)
    cparams = pltpu.CompilerParams(use_tc_tiling_on_sc=False)
    out_type = [jax.ShapeDtypeStruct((_NC, _N_PAD, width), jnp.float32)]
    if with_counts:
        out_type.append(jax.ShapeDtypeStruct((_NC, _N_PAD, 16), jnp.float32))
    scratch = [
        pltpu.VMEM((_G, _CH), jnp.int32),
        pltpu.VMEM((_G, _CH), jnp.int32),
    ]
    nbuf = 2 if with_counts else 4
    scratch += [pltpu.VMEM((_CH, width), jnp.float32)] * nbuf
    scratch += [
        pltpu.VMEM_SHARED((_N_PAD, width), jnp.float32),
        pltpu.SemaphoreType.DMA,
        pltpu.SemaphoreType.DMA,
    ]
    if with_counts:
        scratch += [
            pltpu.VMEM((_CH, 16), jnp.float32),
            pltpu.VMEM_SHARED((_N_PAD, 16), jnp.float32),
            pltpu.SemaphoreType.DMA,
        ]

    def _group_body(feats, src_v, dst_v, bufs, acc_sh, sem_g, sem_s,
                    ones_v, cnt_sh, sem_c):
        nb = len(bufs)
        dg = [None] * _G
        ds = [None] * _G
        dc = [None] * _G
        for j in range(min(nb - 1, _G)):
            dg[j] = pltpu.async_copy(feats.at[src_v.at[j]], bufs[j % nb],
                                     sem_g)
        for j in range(_G):
            if j >= 1:
                ds[j - 1].wait()
                if with_counts:
                    dc[j - 1].wait()
            nxt = j + nb - 1
            if nxt < _G:
                dg[nxt] = pltpu.async_copy(feats.at[src_v.at[nxt]],
                                           bufs[nxt % nb], sem_g)
            dg[j].wait()
            ds[j] = pltpu.async_copy(bufs[j % nb], acc_sh.at[dst_v.at[j]],
                                     sem_s, add=True)
            if with_counts:
                dc[j] = pltpu.async_copy(ones_v, cnt_sh.at[dst_v.at[j]],
                                         sem_c, add=True)
        ds[_G - 1].wait()
        if with_counts:
            dc[_G - 1].wait()

    if with_counts:
        @functools.partial(pl.kernel, out_type=out_type, mesh=mesh,
                           scratch_types=scratch, compiler_params=cparams)
        def kern(feats, src3, dst3, zrow, zcnt,
                 s_out, c_out, src_v, dst_v, *rest):
            bufs = rest[:nbuf]
            acc_sh, sem_g, sem_s, ones_v, cnt_sh, sem_c = rest[nbuf:]
            cid = lax.axis_index("c")
            sid = lax.axis_index("s")
            rows = pl.ds(sid * _RPT, _RPT)
            pltpu.sync_copy(zrow, acc_sh.at[rows])
            pltpu.sync_copy(zcnt, cnt_sh.at[rows])
            for i in range(_CH):
                ones_v[i, :] = jnp.full((16,), 1.0, jnp.float32)
            plsc.subcore_barrier()
            is_fast = cid == _FAST_CID
            g_lo = jnp.where(is_fast, 0, _NGF)
            g_hi = jnp.where(is_fast, _NGF, _NGT)

            @pl.loop(g_lo, g_hi)
            def _(g):
                pltpu.sync_copy(src3.at[sid, pl.ds(g * _G, _G)], src_v)
                pltpu.sync_copy(dst3.at[sid, pl.ds(g * _G, _G)], dst_v)
                _group_body(feats, src_v, dst_v, bufs, acc_sh,
                            sem_g, sem_s, ones_v, cnt_sh, sem_c)

            plsc.subcore_barrier()
            pltpu.sync_copy(acc_sh.at[rows], s_out.at[cid, rows])
            pltpu.sync_copy(cnt_sh.at[rows], c_out.at[cid, rows])
        return kern

    @functools.partial(pl.kernel, out_type=out_type, mesh=mesh,
                       scratch_types=scratch, compiler_params=cparams)
    def kern2(feats, src3, dst3, zrow,
              s_out, src_v, dst_v, *rest):
        bufs = rest[:nbuf]
        acc_sh, sem_g, sem_s = rest[nbuf:]
        cid = lax.axis_index("c")
        sid = lax.axis_index("s")
        rows = pl.ds(sid * _RPT, _RPT)
        pltpu.sync_copy(zrow, acc_sh.at[rows])
        plsc.subcore_barrier()
        is_fast = cid == _FAST_CID
        g_lo = jnp.where(is_fast, 0, _NGF)
        g_hi = jnp.where(is_fast, _NGF, _NGT)

        @pl.loop(g_lo, g_hi)
        def _(g):
            pltpu.sync_copy(src3.at[sid, pl.ds(g * _G, _G)], src_v)
            pltpu.sync_copy(dst3.at[sid, pl.ds(g * _G, _G)], dst_v)
            _group_body(feats, src_v, dst_v, bufs, acc_sh,
                        sem_g, sem_s, None, None, None)

        plsc.subcore_barrier()
        pltpu.sync_copy(acc_sh.at[rows], s_out.at[cid, rows])
    return kern2


_sc_layer1 = _sc_scatter(_D, True)
_sc_layer2 = _sc_scatter(_W2, False)


def _tc_mid(s1, cnt, x, W1l, b1, W1r, W2lp, b2p, W2rp):
    blk = 1000
    grid = _N // blk

    def body(s_ref, c_ref, x_ref, w1l, b1r, w1r, w2l, b2r, w2r,
             p2_ref, r2_ref):
        ssum = s_ref[0] + s_ref[1]
        csum = jnp.maximum(c_ref[0] + c_ref[1], 1.0)[:, 0:1]
        mean1 = ssum / csum
        xb = x_ref[...]
        h1 = mean1 @ w1l[...].T + b1r[...] + xb @ w1r[...].T
        h1 = jnp.maximum(h1, 0.0)
        p2_ref[...] = h1 @ w2l[...].T
        r2_ref[...] = h1 @ w2r[...].T + b2r[...]

    return pl.pallas_call(
        body,
        grid=(grid,),
        in_specs=[
            pl.BlockSpec((_NC, blk, _D), lambda i: (0, i, 0)),
            pl.BlockSpec((_NC, blk, 16), lambda i: (0, i, 0)),
            pl.BlockSpec((blk, _D), lambda i: (i, 0)),
            pl.BlockSpec((_D, _D), lambda i: (0, 0)),
            pl.BlockSpec((1, _D), lambda i: (0, 0)),
            pl.BlockSpec((_D, _D), lambda i: (0, 0)),
            pl.BlockSpec((_W2, _D), lambda i: (0, 0)),
            pl.BlockSpec((1, _W2), lambda i: (0, 0)),
            pl.BlockSpec((_W2, _D), lambda i: (0, 0)),
        ],
        out_specs=[
            pl.BlockSpec((blk, _W2), lambda i: (i, 0)),
            pl.BlockSpec((blk, _W2), lambda i: (i, 0)),
        ],
        out_shape=[
            jax.ShapeDtypeStruct((_N, _W2), jnp.float32),
            jax.ShapeDtypeStruct((_N, _W2), jnp.float32),
        ],
    )(s1, cnt, x, W1l, b1, W1r, W2lp, b2p, W2rp)


def _tc_final(s2, cnt, r2):
    blk = 1000
    grid = _N // blk

    def body(s_ref, c_ref, r_ref, o_ref):
        ssum = s_ref[0] + s_ref[1]
        csum = jnp.maximum(c_ref[0] + c_ref[1], 1.0)[:, 0:1]
        o_ref[...] = ssum / csum + r_ref[...]

    return pl.pallas_call(
        body,
        grid=(grid,),
        in_specs=[
            pl.BlockSpec((_NC, blk, _W2), lambda i: (0, i, 0)),
            pl.BlockSpec((_NC, blk, 16), lambda i: (0, i, 0)),
            pl.BlockSpec((blk, _W2), lambda i: (i, 0)),
        ],
        out_specs=pl.BlockSpec((blk, _W2), lambda i: (i, 0)),
        out_shape=jax.ShapeDtypeStruct((_N, _W2), jnp.float32),
    )(s2, cnt, r2)


def kernel(x, edge_index, W1l, b1, W1r, W2l, b2, W2r):
    src = edge_index[0].astype(jnp.int32)
    dst = edge_index[1].astype(jnp.int32)
    pad = _E_PAD - _E
    src3 = jnp.concatenate([src, jnp.zeros((pad,), jnp.int32)]
                           ).reshape(_NS, _KT, _CH)
    dst3 = jnp.concatenate([dst, jnp.full((pad,), _N, jnp.int32)]
                           ).reshape(_NS, _KT, _CH)
    zrow = jnp.zeros((_RPT, _D), jnp.float32)
    zrow2 = jnp.zeros((_RPT, _W2), jnp.float32)
    zcnt = jnp.zeros((_RPT, 16), jnp.float32)

    s1, cnt = _sc_layer1(x, src3, dst3, zrow, zcnt)

    W2lp = jnp.zeros((_W2, _D), jnp.float32).at[:_NCLS].set(W2l)
    W2rp = jnp.zeros((_W2, _D), jnp.float32).at[:_NCLS].set(W2r)
    b2p = jnp.zeros((1, _W2), jnp.float32).at[0, :_NCLS].set(b2)

    p2, r2 = _tc_mid(s1, cnt, x, W1l, b1.reshape(1, _D), W1r,
                     W2lp, b2p, W2rp)

    [s2] = _sc_layer2(p2, src3, dst3, zrow2)

    out = _tc_final(s2, cnt, r2)
    return out[:, :_NCLS]
